# baseline (speedup 1.0000x reference)
import functools

import jax
import jax.numpy as jnp
from jax import lax
from jax.experimental import pallas as pl
from jax.experimental.pallas import tpu as pltpu
from jax.experimental.pallas import tpu_sc as plsc

_FIELD_SIZE = 100000
_NUM_FIELDS = 26
_EMBED_DIM = 16
_LANES = 16

_NC = 2
_NS = 16
_NW = _NC * _NS

_VOCAB = _FIELD_SIZE * _NUM_FIELDS
_FULL_COLS = _VOCAB // 128
_R_ROWS = (_FULL_COLS + 1) * 16

_mesh = lambda: plsc.VectorSubcoreMesh(core_axis_name="c", subcore_axis_name="s")
_params = pltpu.CompilerParams(
    use_tc_tiling_on_sc=True, needs_layout_passes=False
)


def _transpose_block(lane, cbases, sbases, s_ref, o_ref, n_r=16):
    for r in range(n_r):
        row = jnp.full((_LANES,), r, jnp.int32)
        for m in range(8):
            v = plsc.load_gather(s_ref, [lane, cbases[m] + 8 * r])
            plsc.store_scatter(o_ref, [row, sbases[m]], v)


@functools.partial(
    pl.kernel,
    out_type=jax.ShapeDtypeStruct((_R_ROWS, 128), jnp.float32),
    mesh=_mesh(),
    scratch_types=[
        pltpu.VMEM((_EMBED_DIM, 256), jnp.float32),
        pltpu.VMEM((_EMBED_DIM, 256), jnp.float32),
        pltpu.VMEM((32, 128), jnp.float32),
        pltpu.VMEM((32, 128), jnp.float32),
        pltpu.VMEM((_EMBED_DIM, 128), jnp.float32),
        pltpu.VMEM((8, 128), jnp.float32),
        pltpu.SemaphoreType.DMA,
        pltpu.SemaphoreType.DMA,
        pltpu.SemaphoreType.DMA,
        pltpu.SemaphoreType.DMA,
    ],
    compiler_params=_params,
)
def _repack_kernel(tbl_hbm, tail_hbm, r_hbm, s0, s1, o0, o1, st, ot, i0, i1, q0, q1):
    wid = lax.axis_index("s") * _NC + lax.axis_index("c")
    lane = lax.broadcasted_iota(jnp.int32, (_LANES,), 0)
    cbases = [lax.rem(lane + m, 8) for m in range(8)]
    sbases = [lax.rem(lane + m, 8) * _LANES + lane for m in range(8)]

    pltpu.async_copy(tbl_hbm.at[:, pl.ds(wid * 256, 256)], s0, i0)
    pltpu.async_copy(tbl_hbm.at[:, pl.ds((wid + 32) * 256, 256)], s1, i1)

    def body(i, _):
        for s, o, isem, qsem, off in (
            (s0, o0, i0, q0, 0),
            (s1, o1, i1, q1, 32),
        ):
            c = wid + 64 * i + off

            @pl.when(c < _FULL_COLS // 2)
            def _(s=s, o=o, isem=isem, qsem=qsem, c=c):
                pltpu.make_async_copy(
                    tbl_hbm.at[:, pl.ds(0, 256)], s, isem
                ).wait()

                @pl.when(i > 0)
                def _():
                    pltpu.make_async_copy(
                        o, r_hbm.at[pl.ds(0, 32)], qsem
                    ).wait()

                _transpose_block(lane, cbases, sbases, s, o, n_r=32)

                @pl.when(c + 64 < _FULL_COLS // 2)
                def _():
                    pltpu.async_copy(
                        tbl_hbm.at[:, pl.ds((c + 64) * 256, 256)], s, isem
                    )

                pltpu.async_copy(o, r_hbm.at[pl.ds(c * 32, 32)], qsem)

        return 0

    lax.fori_loop(0, _FULL_COLS // 2 // 64 + 1, body, 0)
    pltpu.make_async_copy(o0, r_hbm.at[pl.ds(0, 32)], q0).wait()
    pltpu.make_async_copy(o1, r_hbm.at[pl.ds(0, 32)], q1).wait()

    @pl.when(wid == 0)
    def _():
        pltpu.sync_copy(tail_hbm, st)
        _transpose_block(lane, cbases, sbases, st, ot, n_r=8)
        pltpu.sync_copy(ot, r_hbm.at[pl.ds(_FULL_COLS * 16, 8)])


def _make_lookup(n_rows):
    b_per_w = n_rows // _NW // _NUM_FIELDS
    chunk = 208
    b_per_chunk = chunk // _NUM_FIELDS
    n_chunks = b_per_w * _NUM_FIELDS // chunk
    groups = chunk // _LANES

    @functools.partial(
        pl.kernel,
        out_type=jax.ShapeDtypeStruct(
            (_NUM_FIELDS, _EMBED_DIM, n_rows // _NUM_FIELDS), jnp.float32
        ),
        mesh=_mesh(),
        scratch_types=[
            pltpu.VMEM((b_per_w * _NUM_FIELDS,), jnp.int32),
            pltpu.VMEM((chunk,), jnp.int32),
            pltpu.VMEM((chunk,), jnp.int32),
            pltpu.VMEM((chunk, 128), jnp.float32),
            pltpu.VMEM((chunk, 128), jnp.float32),
            pltpu.VMEM((_NUM_FIELDS, _EMBED_DIM, 128), jnp.float32),
            pltpu.SemaphoreType.DMA,
            pltpu.SemaphoreType.DMA,
        ],
        compiler_params=_params,
    )
    def lookup_kernel(x_hbm, r_hbm, out_hbm, abuf, jb0, jb1, rb0, rb1,
                      tacc, g0, g1):
        wid = lax.axis_index("s") * _NC + lax.axis_index("c")
        base = wid * b_per_w * _NUM_FIELDS
        lane = lax.broadcasted_iota(jnp.int32, (_LANES,), 0)

        pltpu.sync_copy(x_hbm.at[pl.ds(base, b_per_w * _NUM_FIELDS)], abuf)

        def adj_body(j, _):
            for u in range(4):
                sl = pl.ds((j * 4 + u) * _LANES, _LANES)
                pos = (j * 4 + u) * _LANES + lane
                abuf[sl] = abuf[sl] + lax.rem(pos, _NUM_FIELDS) * _FIELD_SIZE
            return 0

        lax.fori_loop(0, b_per_w * _NUM_FIELDS // _LANES // 4, adj_body, 0)

        def fill(cc, jb):
            for g in range(groups):
                av = abuf[pl.ds(cc * chunk + g * _LANES, _LANES)]
                jb[pl.ds(g * _LANES, _LANES)] = lax.shift_right_logical(
                    av, 3
                )

        def select(cc, rb):
            blk = lax.rem(cc, 16)
            for g in range(groups):
                av = abuf[pl.ds(cc * chunk + g * _LANES, _LANES)]
                col_base = lax.bitwise_and(av, 7) * _EMBED_DIM
                row = lane + g * _LANES
                f_vec = lax.rem(g * _LANES + lane, _NUM_FIELDS)
                b_vec = (g * _LANES + lane) // _NUM_FIELDS + blk * b_per_chunk

                def e_body(e, _, rb=rb, row=row, col_base=col_base,
                           f_vec=f_vec, b_vec=b_vec):
                    v = plsc.load_gather(rb, [row, col_base + e])
                    e_vec = jnp.full((_LANES,), e, jnp.int32)
                    plsc.store_scatter(tacc, [f_vec, e_vec, b_vec], v)
                    return 0

                lax.fori_loop(0, _EMBED_DIM, e_body, 0)

        def gather(cc, jb, rb, sem):
            fill(cc, jb)
            return pltpu.async_copy(r_hbm.at[jb], rb, sem)

        gather(0, jb0, rb0, g0)
        gather(1, jb1, rb1, g1)

        def body(p, _):
            for jb, rb, sem, off in ((jb0, rb0, g0, 0), (jb1, rb1, g1, 1)):
                cc = 2 * p + off
                pltpu.make_async_copy(r_hbm.at[jb], rb, sem).wait()
                select(cc, rb)

                @pl.when(cc + 2 < n_chunks)
                def _(cc=cc, jb=jb, rb=rb, sem=sem):
                    gather(cc + 2, jb, rb, sem)

                @pl.when(lax.rem(cc, 16) == 15)
                def _(cc=cc):
                    pltpu.sync_copy(
                        tacc,
                        out_hbm.at[
                            :, :, pl.ds(wid * b_per_w + (cc // 16) * 128, 128)
                        ],
                    )

            return 0

        lax.fori_loop(0, n_chunks // 2, body, 0)

    return lookup_kernel


def kernel(x, table):
    b, f = x.shape
    n_rows = b * f
    x_flat = x.reshape(n_rows).astype(jnp.int32)
    table = table.astype(jnp.float32)
    tbl_t = table.T
    tail = jnp.pad(table[_FULL_COLS * 128:, :], ((0, 64), (0, 0))).T
    r = _repack_kernel(tbl_t, tail)
    out = _make_lookup(n_rows)(x_flat, r)
    return out.transpose(2, 0, 1)


# speedup vs baseline: 1.2790x; 1.2790x over previous
import functools

import jax
import jax.numpy as jnp
from jax import lax
from jax.experimental import pallas as pl
from jax.experimental.pallas import tpu as pltpu
from jax.experimental.pallas import tpu_sc as plsc

_FIELD_SIZE = 100000
_NUM_FIELDS = 26
_EMBED_DIM = 16
_LANES = 16

_NC = 2
_NS = 16
_NW = _NC * _NS

_VOCAB = _FIELD_SIZE * _NUM_FIELDS
_FULL_COLS = _VOCAB // 128
_R_ROWS = (_FULL_COLS + 1) * 16

_mesh = lambda: plsc.VectorSubcoreMesh(core_axis_name="c", subcore_axis_name="s")
_params = pltpu.CompilerParams(
    use_tc_tiling_on_sc=True, needs_layout_passes=False
)


def _transpose_block(lane, cbases, sbases, s_ref, o_ref, n_r=16):
    for r in range(n_r):
        row = jnp.full((_LANES,), r, jnp.int32)
        for m in range(8):
            v = plsc.load_gather(s_ref, [lane, cbases[m] + 8 * r])
            plsc.store_scatter(o_ref, [row, sbases[m]], v)


@functools.partial(
    pl.kernel,
    out_type=jax.ShapeDtypeStruct((_R_ROWS, 128), jnp.float32),
    mesh=_mesh(),
    scratch_types=[
        pltpu.VMEM((_EMBED_DIM, 128), jnp.float32),
        pltpu.VMEM((_EMBED_DIM, 128), jnp.float32),
        pltpu.VMEM((16, 128), jnp.float32),
        pltpu.VMEM((16, 128), jnp.float32),
        pltpu.SemaphoreType.DMA,
        pltpu.SemaphoreType.DMA,
        pltpu.SemaphoreType.DMA,
        pltpu.SemaphoreType.DMA,
    ],
    compiler_params=_params,
)
def _repack_kernel(tbl_hbm, tail_hbm, r_hbm, s0, s1, o0, o1, i0, i1, q0, q1):
    wid = lax.axis_index("s") * _NC + lax.axis_index("c")
    lane = lax.broadcasted_iota(jnp.int32, (_LANES,), 0)
    cbases = [lax.rem(lane + m, 8) for m in range(8)]
    sbases = [lax.rem(lane + m, 8) * _LANES + lane for m in range(8)]

    pltpu.async_copy(tbl_hbm.at[:, pl.ds(wid * 128, 128)], s0, i0)
    pltpu.async_copy(tbl_hbm.at[:, pl.ds((wid + 32) * 128, 128)], s1, i1)

    def body(i, _):
        for s, o, isem, qsem, off in (
            (s0, o0, i0, q0, 0),
            (s1, o1, i1, q1, 32),
        ):
            c = wid + 64 * i + off

            @pl.when(c < _FULL_COLS)
            def _(s=s, o=o, isem=isem, qsem=qsem, c=c):
                pltpu.make_async_copy(
                    tbl_hbm.at[:, pl.ds(0, 128)], s, isem
                ).wait()

                @pl.when(i > 0)
                def _():
                    pltpu.make_async_copy(
                        o, r_hbm.at[pl.ds(0, 16)], qsem
                    ).wait()

                _transpose_block(lane, cbases, sbases, s, o)

                @pl.when(c + 64 < _FULL_COLS)
                def _():
                    pltpu.async_copy(
                        tbl_hbm.at[:, pl.ds((c + 64) * 128, 128)], s, isem
                    )

                pltpu.async_copy(o, r_hbm.at[pl.ds(c * 16, 16)], qsem)

        return 0

    lax.fori_loop(0, _FULL_COLS // 64 + 1, body, 0)
    pltpu.make_async_copy(o0, r_hbm.at[pl.ds(0, 16)], q0).wait()
    pltpu.make_async_copy(o1, r_hbm.at[pl.ds(0, 16)], q1).wait()

    @pl.when(wid == 0)
    def _():
        pltpu.sync_copy(tail_hbm, s0)
        _transpose_block(lane, cbases, sbases, s0, o0)
        pltpu.sync_copy(o0, r_hbm.at[pl.ds(_FULL_COLS * 16, 16)])


def _make_lookup(n_rows):
    b_per_w = n_rows // _NW // _NUM_FIELDS
    chunk = 208
    b_per_chunk = chunk // _NUM_FIELDS
    n_chunks = b_per_w * _NUM_FIELDS // chunk
    groups = chunk // _LANES

    @functools.partial(
        pl.kernel,
        out_type=jax.ShapeDtypeStruct(
            (_NUM_FIELDS, _EMBED_DIM, n_rows // _NUM_FIELDS), jnp.float32
        ),
        mesh=_mesh(),
        scratch_types=[
            pltpu.VMEM((b_per_w * _NUM_FIELDS,), jnp.int32),
            pltpu.VMEM((chunk,), jnp.int32),
            pltpu.VMEM((chunk,), jnp.int32),
            pltpu.VMEM((chunk, 128), jnp.float32),
            pltpu.VMEM((chunk, 128), jnp.float32),
            pltpu.VMEM((_NUM_FIELDS, _EMBED_DIM, 128), jnp.float32),
            pltpu.SemaphoreType.DMA,
            pltpu.SemaphoreType.DMA,
        ],
        compiler_params=_params,
    )
    def lookup_kernel(x_hbm, r_hbm, out_hbm, abuf, jb0, jb1, rb0, rb1,
                      tacc, g0, g1):
        wid = lax.axis_index("s") * _NC + lax.axis_index("c")
        base = wid * b_per_w * _NUM_FIELDS
        lane = lax.broadcasted_iota(jnp.int32, (_LANES,), 0)

        pltpu.sync_copy(x_hbm.at[pl.ds(base, b_per_w * _NUM_FIELDS)], abuf)

        def adj_body(j, _):
            for u in range(4):
                sl = pl.ds((j * 4 + u) * _LANES, _LANES)
                pos = (j * 4 + u) * _LANES + lane
                abuf[sl] = abuf[sl] + lax.rem(pos, _NUM_FIELDS) * _FIELD_SIZE
            return 0

        lax.fori_loop(0, b_per_w * _NUM_FIELDS // _LANES // 4, adj_body, 0)

        def fill(cc, jb):
            for g in range(groups):
                av = abuf[pl.ds(cc * chunk + g * _LANES, _LANES)]
                jb[pl.ds(g * _LANES, _LANES)] = lax.shift_right_logical(
                    av, 3
                )

        def select(cc, rb):
            blk = lax.rem(cc, 16)
            for g in range(groups):
                av = abuf[pl.ds(cc * chunk + g * _LANES, _LANES)]
                col_base = lax.bitwise_and(av, 7) * _EMBED_DIM
                row = lane + g * _LANES
                f_vec = lax.rem(g * _LANES + lane, _NUM_FIELDS)
                b_vec = (g * _LANES + lane) // _NUM_FIELDS + blk * b_per_chunk

                def e_body(e, _, rb=rb, row=row, col_base=col_base,
                           f_vec=f_vec, b_vec=b_vec):
                    v = plsc.load_gather(rb, [row, col_base + e])
                    e_vec = jnp.full((_LANES,), e, jnp.int32)
                    plsc.store_scatter(tacc, [f_vec, e_vec, b_vec], v)
                    return 0

                lax.fori_loop(0, _EMBED_DIM, e_body, 0)

        def gather(cc, jb, rb, sem):
            fill(cc, jb)
            return pltpu.async_copy(r_hbm.at[jb], rb, sem)

        gather(0, jb0, rb0, g0)
        gather(1, jb1, rb1, g1)

        def body(p, _):
            for jb, rb, sem, off in ((jb0, rb0, g0, 0), (jb1, rb1, g1, 1)):
                cc = 2 * p + off
                pltpu.make_async_copy(r_hbm.at[jb], rb, sem).wait()
                select(cc, rb)

                @pl.when(cc + 2 < n_chunks)
                def _(cc=cc, jb=jb, rb=rb, sem=sem):
                    gather(cc + 2, jb, rb, sem)

                @pl.when(lax.rem(cc, 16) == 15)
                def _(cc=cc):
                    pltpu.sync_copy(
                        tacc,
                        out_hbm.at[
                            :, :, pl.ds(wid * b_per_w + (cc // 16) * 128, 128)
                        ],
                    )

            return 0

        lax.fori_loop(0, n_chunks // 2, body, 0)

    return lookup_kernel


def kernel(x, table):
    b, f = x.shape
    n_rows = b * f
    x_flat = x.reshape(n_rows).astype(jnp.int32)
    table = table.astype(jnp.float32)
    tbl_t = table.T
    tail = jnp.pad(table[_FULL_COLS * 128:, :], ((0, 64), (0, 0))).T
    r = _repack_kernel(tbl_t, tail)
    out = _make_lookup(n_rows)(x_flat, r)
    return out.transpose(2, 0, 1)


# speedup vs baseline: 1.3606x; 1.0638x over previous
import functools

import jax
import jax.numpy as jnp
from jax import lax
from jax.experimental import pallas as pl
from jax.experimental.pallas import tpu as pltpu
from jax.experimental.pallas import tpu_sc as plsc

_FIELD_SIZE = 100000
_NUM_FIELDS = 26
_EMBED_DIM = 16
_LANES = 16

_NC = 2
_NS = 16
_NW = _NC * _NS

_VOCAB = _FIELD_SIZE * _NUM_FIELDS
_FULL_COLS = _VOCAB // 128
_R_ROWS = (_FULL_COLS + 1) * 16

_mesh = lambda: plsc.VectorSubcoreMesh(core_axis_name="c", subcore_axis_name="s")
_params = pltpu.CompilerParams(
    use_tc_tiling_on_sc=True, needs_layout_passes=False
)


def _transpose_block(lane, cbases, sbases, s_ref, o_ref, n_r=16):
    for r in range(n_r):
        row = jnp.full((_LANES,), r, jnp.int32)
        vs = [
            plsc.load_gather(s_ref, [lane, cbases[m] + 8 * r])
            for m in range(8)
        ]
        for m in range(8):
            plsc.store_scatter(o_ref, [row, sbases[m]], vs[m])


@functools.partial(
    pl.kernel,
    out_type=jax.ShapeDtypeStruct((_R_ROWS, 128), jnp.float32),
    mesh=_mesh(),
    scratch_types=[
        pltpu.VMEM((_EMBED_DIM, 128), jnp.float32),
        pltpu.VMEM((_EMBED_DIM, 128), jnp.float32),
        pltpu.VMEM((16, 128), jnp.float32),
        pltpu.VMEM((16, 128), jnp.float32),
        pltpu.SemaphoreType.DMA,
        pltpu.SemaphoreType.DMA,
        pltpu.SemaphoreType.DMA,
        pltpu.SemaphoreType.DMA,
    ],
    compiler_params=_params,
)
def _repack_kernel(tbl_hbm, tail_hbm, r_hbm, s0, s1, o0, o1, i0, i1, q0, q1):
    wid = lax.axis_index("s") * _NC + lax.axis_index("c")
    lane = lax.broadcasted_iota(jnp.int32, (_LANES,), 0)
    cbases = [lax.rem(lane + m, 8) for m in range(8)]
    sbases = [lax.rem(lane + m, 8) * _LANES + lane for m in range(8)]

    pltpu.async_copy(tbl_hbm.at[:, pl.ds(wid * 128, 128)], s0, i0)
    pltpu.async_copy(tbl_hbm.at[:, pl.ds((wid + 32) * 128, 128)], s1, i1)

    def body(i, _):
        for s, o, isem, qsem, off in (
            (s0, o0, i0, q0, 0),
            (s1, o1, i1, q1, 32),
        ):
            c = wid + 64 * i + off

            @pl.when(c < _FULL_COLS)
            def _(s=s, o=o, isem=isem, qsem=qsem, c=c):
                pltpu.make_async_copy(
                    tbl_hbm.at[:, pl.ds(0, 128)], s, isem
                ).wait()

                @pl.when(i > 0)
                def _():
                    pltpu.make_async_copy(
                        o, r_hbm.at[pl.ds(0, 16)], qsem
                    ).wait()

                _transpose_block(lane, cbases, sbases, s, o)

                @pl.when(c + 64 < _FULL_COLS)
                def _():
                    pltpu.async_copy(
                        tbl_hbm.at[:, pl.ds((c + 64) * 128, 128)], s, isem
                    )

                pltpu.async_copy(o, r_hbm.at[pl.ds(c * 16, 16)], qsem)

        return 0

    lax.fori_loop(0, _FULL_COLS // 64 + 1, body, 0)
    pltpu.make_async_copy(o0, r_hbm.at[pl.ds(0, 16)], q0).wait()
    pltpu.make_async_copy(o1, r_hbm.at[pl.ds(0, 16)], q1).wait()

    @pl.when(wid == 0)
    def _():
        pltpu.sync_copy(tail_hbm, s0)
        _transpose_block(lane, cbases, sbases, s0, o0)
        pltpu.sync_copy(o0, r_hbm.at[pl.ds(_FULL_COLS * 16, 16)])


def _make_lookup(n_rows):
    b_per_w = n_rows // _NW // _NUM_FIELDS
    chunk = 208
    b_per_chunk = chunk // _NUM_FIELDS
    n_chunks = b_per_w * _NUM_FIELDS // chunk
    groups = chunk // _LANES

    @functools.partial(
        pl.kernel,
        out_type=jax.ShapeDtypeStruct(
            (_NUM_FIELDS, _EMBED_DIM, n_rows // _NUM_FIELDS), jnp.float32
        ),
        mesh=_mesh(),
        scratch_types=[
            pltpu.VMEM((b_per_w * _NUM_FIELDS,), jnp.int32),
            pltpu.VMEM((chunk,), jnp.int32),
            pltpu.VMEM((chunk,), jnp.int32),
            pltpu.VMEM((chunk, 128), jnp.float32),
            pltpu.VMEM((chunk, 128), jnp.float32),
            pltpu.VMEM((_NUM_FIELDS, _EMBED_DIM, 128), jnp.float32),
            pltpu.SemaphoreType.DMA,
            pltpu.SemaphoreType.DMA,
        ],
        compiler_params=_params,
    )
    def lookup_kernel(x_hbm, r_hbm, out_hbm, abuf, jb0, jb1, rb0, rb1,
                      tacc, g0, g1):
        wid = lax.axis_index("s") * _NC + lax.axis_index("c")
        base = wid * b_per_w * _NUM_FIELDS
        lane = lax.broadcasted_iota(jnp.int32, (_LANES,), 0)

        pltpu.sync_copy(x_hbm.at[pl.ds(base, b_per_w * _NUM_FIELDS)], abuf)

        def adj_body(j, _):
            for u in range(4):
                sl = pl.ds((j * 4 + u) * _LANES, _LANES)
                pos = (j * 4 + u) * _LANES + lane
                abuf[sl] = abuf[sl] + lax.rem(pos, _NUM_FIELDS) * _FIELD_SIZE
            return 0

        lax.fori_loop(0, b_per_w * _NUM_FIELDS // _LANES // 4, adj_body, 0)

        def fill(cc, jb):
            for g in range(groups):
                av = abuf[pl.ds(cc * chunk + g * _LANES, _LANES)]
                jb[pl.ds(g * _LANES, _LANES)] = lax.shift_right_logical(
                    av, 3
                )

        def select(cc, rb):
            blk = lax.rem(cc, 16)
            for g in range(groups):
                av = abuf[pl.ds(cc * chunk + g * _LANES, _LANES)]
                col_base = lax.bitwise_and(av, 7) * _EMBED_DIM
                row = lane + g * _LANES
                f_vec = lax.rem(g * _LANES + lane, _NUM_FIELDS)
                b_vec = (g * _LANES + lane) // _NUM_FIELDS + blk * b_per_chunk

                def e_body(e, _, rb=rb, row=row, col_base=col_base,
                           f_vec=f_vec, b_vec=b_vec):
                    v = plsc.load_gather(rb, [row, col_base + e])
                    e_vec = jnp.full((_LANES,), e, jnp.int32)
                    plsc.store_scatter(tacc, [f_vec, e_vec, b_vec], v)
                    return 0

                lax.fori_loop(0, _EMBED_DIM, e_body, 0)

        def gather(cc, jb, rb, sem):
            fill(cc, jb)
            return pltpu.async_copy(r_hbm.at[jb], rb, sem)

        gather(0, jb0, rb0, g0)
        gather(1, jb1, rb1, g1)

        def body(p, _):
            for jb, rb, sem, off in ((jb0, rb0, g0, 0), (jb1, rb1, g1, 1)):
                cc = 2 * p + off
                pltpu.make_async_copy(r_hbm.at[jb], rb, sem).wait()
                select(cc, rb)

                @pl.when(cc + 2 < n_chunks)
                def _(cc=cc, jb=jb, rb=rb, sem=sem):
                    gather(cc + 2, jb, rb, sem)

                @pl.when(lax.rem(cc, 16) == 15)
                def _(cc=cc):
                    pltpu.sync_copy(
                        tacc,
                        out_hbm.at[
                            :, :, pl.ds(wid * b_per_w + (cc // 16) * 128, 128)
                        ],
                    )

            return 0

        lax.fori_loop(0, n_chunks // 2, body, 0)

    return lookup_kernel


def kernel(x, table):
    b, f = x.shape
    n_rows = b * f
    x_flat = x.reshape(n_rows).astype(jnp.int32)
    table = table.astype(jnp.float32)
    tbl_t = table.T
    tail = jnp.pad(table[_FULL_COLS * 128:, :], ((0, 64), (0, 0))).T
    r = _repack_kernel(tbl_t, tail)
    out = _make_lookup(n_rows)(x_flat, r)
    return out.transpose(2, 0, 1)
